# baseline (speedup 1.0000x reference)
import functools

import jax
import jax.numpy as jnp
from jax import lax
from jax.experimental import pallas as pl
from jax.experimental.pallas import tpu as pltpu
from jax.experimental.pallas import tpu_sc as plsc

N_NODES = 10000
N_EDGES = 320000
C = 128

NC = 2
NS = 16
NW = NC * NS

W = 64
EDGES_PER_W = (N_EDGES // NW // W + 1) * W
CHUNKS_PER_W = EDGES_PER_W // W
N_PAD = 10240
NODE_CHUNK = N_PAD // NS


def _sc_aggregate(x, sd_slab):
  @functools.partial(
      pl.kernel,
      mesh=plsc.VectorSubcoreMesh(core_axis_name="c", subcore_axis_name="s"),
      compiler_params=pltpu.CompilerParams(use_tc_tiling_on_sc=False),
      out_type=[
          jax.ShapeDtypeStruct((NC, N_PAD, C), jnp.float32),
          jax.ShapeDtypeStruct((NC, N_PAD, 16), jnp.float32),
      ],
      scratch_types=[
          pltpu.VMEM((2 * CHUNKS_PER_W, W), jnp.int32),
          pltpu.VMEM((2, W, C), jnp.float32),
          pltpu.VMEM((W, 16), jnp.float32),
          pltpu.VMEM_SHARED((N_PAD, C), jnp.float32),
          pltpu.VMEM_SHARED((N_PAD, 16), jnp.float32),
          pltpu.SemaphoreType.DMA((2,)),
      ],
  )
  def k(x_hbm, sd_hbm, acc_out, cnt_out, slab, rows2, ones_v, acc_sh, cnt_sh,
        sem2):
    c = lax.axis_index("c")
    s = lax.axis_index("s")
    w = s * NC + c

    def fill_zero(i, _):
      def fill_lane(j, _):
        rows2[0, i, pl.ds(j * 16, 16)] = jnp.zeros((16,), jnp.float32)
        return 0
      lax.fori_loop(0, C // 16, fill_lane, 0)
      ones_v[i] = jnp.zeros((16,), jnp.float32)
      return 0
    lax.fori_loop(0, W, fill_zero, 0)

    nb = pl.multiple_of(s * NODE_CHUNK, 8)

    def zero_chunk(j, _):
      off = pl.multiple_of(nb + j * W, 8)
      pltpu.sync_copy(rows2.at[0], acc_sh.at[pl.ds(off, W)])
      pltpu.sync_copy(ones_v, cnt_sh.at[pl.ds(off, W)])
      return 0
    lax.fori_loop(0, NODE_CHUNK // W, zero_chunk, 0)

    def fill_one(i, _):
      ones_v[i] = jnp.ones((16,), jnp.float32)
      return 0
    lax.fori_loop(0, W, fill_one, 0)

    base = pl.multiple_of(2 * CHUNKS_PER_W * w, 8)
    pltpu.sync_copy(sd_hbm.at[pl.ds(base, 2 * CHUNKS_PER_W)], slab)

    plsc.subcore_barrier()

    def step(i, carry):
      b = jnp.bitwise_and(i, 1)

      @pl.when(i < CHUNKS_PER_W)
      def _():
        pltpu.async_copy(x_hbm.at[slab.at[i]], rows2.at[b], sem2.at[b])

      @pl.when(i > 0)
      def _():
        pb = jnp.bitwise_and(i - 1, 1)
        didx = slab.at[CHUNKS_PER_W + i - 1]
        pltpu.make_async_copy(
            x_hbm.at[pl.ds(0, W)], rows2.at[pb], sem2.at[pb]).wait()
        pltpu.sync_copy(rows2.at[pb], acc_sh.at[didx], add=True)
        pltpu.sync_copy(ones_v, cnt_sh.at[didx], add=True)
      return carry

    lax.fori_loop(0, CHUNKS_PER_W + 1, step, 0)

    plsc.subcore_barrier()

    pltpu.sync_copy(acc_sh.at[pl.ds(nb, NODE_CHUNK)],
                    acc_out.at[c, pl.ds(nb, NODE_CHUNK)])
    pltpu.sync_copy(cnt_sh.at[pl.ds(nb, NODE_CHUNK)],
                    cnt_out.at[c, pl.ds(nb, NODE_CHUNK)])

  return k(x, sd_slab)


def _tc_body(pacc, pcnt, x, wl, wr, b, out):
  acc = pacc[0] + pacc[1]
  cnt = pcnt[0] + pcnt[1]
  mean = acc / jnp.maximum(cnt[:, 0:1], 1.0)
  y = (jnp.dot(mean, wl[...], preferred_element_type=jnp.float32)
       + b[...]
       + jnp.dot(x[...], wr[...], preferred_element_type=jnp.float32))
  out[...] = jnp.maximum(y, 0.0)


def _tc_finish(pacc, pcnt, x, wlT, wrT, b):
  R = 2000
  grid = (N_NODES // R,)
  return pl.pallas_call(
      _tc_body,
      grid=grid,
      in_specs=[
          pl.BlockSpec((NC, R, C), lambda i: (0, i, 0)),
          pl.BlockSpec((NC, R, 16), lambda i: (0, i, 0)),
          pl.BlockSpec((R, C), lambda i: (i, 0)),
          pl.BlockSpec((C, C), lambda i: (0, 0)),
          pl.BlockSpec((C, C), lambda i: (0, 0)),
          pl.BlockSpec((1, C), lambda i: (0, 0)),
      ],
      out_specs=pl.BlockSpec((R, C), lambda i: (i, 0)),
      out_shape=jax.ShapeDtypeStruct((N_NODES, C), jnp.float32),
  )(pacc, pcnt, x, wlT, wrT, b)


_N_EDGE_PAD = NW * EDGES_PER_W - N_EDGES


def kernel(x, edge_index, W_l, b_l, W_r):
  ar = jnp.arange(_N_EDGE_PAD, dtype=jnp.int32)
  pad_src = ar % N_NODES
  pad_dst = ar % (N_PAD - N_NODES) + N_NODES
  src3d = jnp.concatenate([edge_index[0], pad_src]).reshape(NW, CHUNKS_PER_W, W)
  dst3d = jnp.concatenate([edge_index[1], pad_dst]).reshape(NW, CHUNKS_PER_W, W)
  sd_slab = jnp.concatenate([src3d, dst3d], axis=1)
  sd_slab = sd_slab.reshape(2 * NW * CHUNKS_PER_W, W)
  pacc, pcnt = _sc_aggregate(x, sd_slab)
  return _tc_finish(pacc, pcnt, x, W_l.T, W_r.T, b_l.reshape(1, C))


# speedup vs baseline: 1.0309x; 1.0309x over previous
import functools

import jax
import jax.numpy as jnp
from jax import lax
from jax.experimental import pallas as pl
from jax.experimental.pallas import tpu as pltpu
from jax.experimental.pallas import tpu_sc as plsc

N_NODES = 10000
N_EDGES = 320000
C = 128

NC = 2
NS = 16
NW = NC * NS

W = 64
EDGES_PER_W = (N_EDGES // NW // W + 1) * W
CHUNKS_PER_W = EDGES_PER_W // W
N_PAD = 10240
NODE_CHUNK = N_PAD // NS


def _sc_aggregate(x, sd_slab):
  @functools.partial(
      pl.kernel,
      mesh=plsc.VectorSubcoreMesh(core_axis_name="c", subcore_axis_name="s"),
      compiler_params=pltpu.CompilerParams(use_tc_tiling_on_sc=False),
      out_type=[
          jax.ShapeDtypeStruct((NC, N_PAD, C), jnp.float32),
          jax.ShapeDtypeStruct((NC, N_PAD, 16), jnp.float32),
      ],
      scratch_types=[
          pltpu.VMEM((2 * CHUNKS_PER_W, W), jnp.int32),
          pltpu.VMEM((2, W, C), jnp.float32),
          pltpu.VMEM((W, 16), jnp.float32),
          pltpu.VMEM_SHARED((N_PAD, C), jnp.float32),
          pltpu.VMEM_SHARED((N_PAD, 16), jnp.float32),
          pltpu.SemaphoreType.DMA((2,)),
          pltpu.SemaphoreType.DMA,
      ],
  )
  def k(x_hbm, sd_hbm, acc_out, cnt_out, slab, rows2, ones_v, acc_sh, cnt_sh,
        sem2, csem):
    c = lax.axis_index("c")
    s = lax.axis_index("s")
    w = s * NC + c

    def fill_zero(i, _):
      def fill_lane(j, _):
        rows2[0, i, pl.ds(j * 16, 16)] = jnp.zeros((16,), jnp.float32)
        return 0
      lax.fori_loop(0, C // 16, fill_lane, 0)
      ones_v[i] = jnp.zeros((16,), jnp.float32)
      return 0
    lax.fori_loop(0, W, fill_zero, 0)

    nb = pl.multiple_of(s * NODE_CHUNK, 8)

    def zero_chunk(j, _):
      off = pl.multiple_of(nb + j * W, 8)
      pltpu.sync_copy(rows2.at[0], acc_sh.at[pl.ds(off, W)])
      pltpu.sync_copy(ones_v, cnt_sh.at[pl.ds(off, W)])
      return 0
    lax.fori_loop(0, NODE_CHUNK // W, zero_chunk, 0)

    def fill_one(i, _):
      ones_v[i] = jnp.ones((16,), jnp.float32)
      return 0
    lax.fori_loop(0, W, fill_one, 0)

    base = pl.multiple_of(2 * CHUNKS_PER_W * w, 8)
    pltpu.sync_copy(sd_hbm.at[pl.ds(base, 2 * CHUNKS_PER_W)], slab)

    plsc.subcore_barrier()

    def step(i, carry):
      b = jnp.bitwise_and(i, 1)

      @pl.when(i < CHUNKS_PER_W)
      def _():
        pltpu.async_copy(x_hbm.at[slab.at[i]], rows2.at[b], sem2.at[b])

      @pl.when(i > 0)
      def _():
        pb = jnp.bitwise_and(i - 1, 1)
        didx = slab.at[CHUNKS_PER_W + i - 1]
        pltpu.make_async_copy(
            x_hbm.at[pl.ds(0, W)], rows2.at[pb], sem2.at[pb]).wait()
        pltpu.sync_copy(rows2.at[pb], acc_sh.at[didx], add=True)
        pltpu.async_copy(ones_v, cnt_sh.at[didx], csem, add=True)
      return carry

    lax.fori_loop(0, CHUNKS_PER_W + 1, step, 0)

    def drain(i, carry):
      pltpu.make_async_copy(
          x_hbm.at[pl.ds(0, W), pl.ds(0, 16)], ones_v, csem).wait()
      return carry

    lax.fori_loop(0, CHUNKS_PER_W, drain, 0)

    plsc.subcore_barrier()

    pltpu.sync_copy(acc_sh.at[pl.ds(nb, NODE_CHUNK)],
                    acc_out.at[c, pl.ds(nb, NODE_CHUNK)])
    pltpu.sync_copy(cnt_sh.at[pl.ds(nb, NODE_CHUNK)],
                    cnt_out.at[c, pl.ds(nb, NODE_CHUNK)])

  return k(x, sd_slab)


def _tc_body(pacc, pcnt, x, wl, wr, b, out):
  acc = pacc[0] + pacc[1]
  cnt = pcnt[0] + pcnt[1]
  mean = acc / jnp.maximum(cnt[:, 0:1], 1.0)
  y = (jnp.dot(mean, wl[...], preferred_element_type=jnp.float32)
       + b[...]
       + jnp.dot(x[...], wr[...], preferred_element_type=jnp.float32))
  out[...] = jnp.maximum(y, 0.0)


def _tc_finish(pacc, pcnt, x, wlT, wrT, b):
  R = 2000
  grid = (N_NODES // R,)
  return pl.pallas_call(
      _tc_body,
      grid=grid,
      in_specs=[
          pl.BlockSpec((NC, R, C), lambda i: (0, i, 0)),
          pl.BlockSpec((NC, R, 16), lambda i: (0, i, 0)),
          pl.BlockSpec((R, C), lambda i: (i, 0)),
          pl.BlockSpec((C, C), lambda i: (0, 0)),
          pl.BlockSpec((C, C), lambda i: (0, 0)),
          pl.BlockSpec((1, C), lambda i: (0, 0)),
      ],
      out_specs=pl.BlockSpec((R, C), lambda i: (i, 0)),
      out_shape=jax.ShapeDtypeStruct((N_NODES, C), jnp.float32),
  )(pacc, pcnt, x, wlT, wrT, b)


_N_EDGE_PAD = NW * EDGES_PER_W - N_EDGES


def kernel(x, edge_index, W_l, b_l, W_r):
  ar = jnp.arange(_N_EDGE_PAD, dtype=jnp.int32)
  pad_src = ar % N_NODES
  pad_dst = ar % (N_PAD - N_NODES) + N_NODES
  src3d = jnp.concatenate([edge_index[0], pad_src]).reshape(NW, CHUNKS_PER_W, W)
  dst3d = jnp.concatenate([edge_index[1], pad_dst]).reshape(NW, CHUNKS_PER_W, W)
  sd_slab = jnp.concatenate([src3d, dst3d], axis=1)
  sd_slab = sd_slab.reshape(2 * NW * CHUNKS_PER_W, W)
  pacc, pcnt = _sc_aggregate(x, sd_slab)
  return _tc_finish(pacc, pcnt, x, W_l.T, W_r.T, b_l.reshape(1, C))


# speedup vs baseline: 1.1409x; 1.1067x over previous
import functools

import jax
import jax.numpy as jnp
from jax import lax
from jax.experimental import pallas as pl
from jax.experimental.pallas import tpu as pltpu
from jax.experimental.pallas import tpu_sc as plsc

N_NODES = 10000
N_EDGES = 320000
C = 128

NC = 2
NS = 16
NW = NC * NS

W = 64
NCHUNK = N_EDGES // W
CPW = NCHUNK // NW
CPW_EXTRA = NCHUNK - CPW * NW
N_PAD = 10240
NODE_CHUNK = N_PAD // NS


def _sc_aggregate(x, e3):
  @functools.partial(
      pl.kernel,
      mesh=plsc.VectorSubcoreMesh(core_axis_name="c", subcore_axis_name="s"),
      compiler_params=pltpu.CompilerParams(use_tc_tiling_on_sc=False),
      out_type=[
          jax.ShapeDtypeStruct((NC, N_PAD, C), jnp.float32),
          jax.ShapeDtypeStruct((NC, N_PAD, 16), jnp.float32),
      ],
      scratch_types=[
          pltpu.VMEM((CPW + 1, W), jnp.int32),
          pltpu.VMEM((CPW + 1, W), jnp.int32),
          pltpu.VMEM((2, W, C), jnp.float32),
          pltpu.VMEM((W, 16), jnp.float32),
          pltpu.VMEM_SHARED((N_PAD, C), jnp.float32),
          pltpu.VMEM_SHARED((N_PAD, 16), jnp.float32),
          pltpu.SemaphoreType.DMA((2,)),
          pltpu.SemaphoreType.DMA,
      ],
  )
  def k(x_hbm, e3_hbm, acc_out, cnt_out, sidx, didx, rows2, ones_v,
        acc_sh, cnt_sh, sem2, csem):
    c = lax.axis_index("c")
    s = lax.axis_index("s")
    w = s * NC + c

    def fill_zero(i, _):
      def fill_lane(j, _):
        rows2[0, i, pl.ds(j * 16, 16)] = jnp.zeros((16,), jnp.float32)
        return 0
      lax.fori_loop(0, C // 16, fill_lane, 0)
      ones_v[i] = jnp.zeros((16,), jnp.float32)
      return 0
    lax.fori_loop(0, W, fill_zero, 0)

    nb = pl.multiple_of(s * NODE_CHUNK, 8)

    def zero_chunk(j, _):
      off = pl.multiple_of(nb + j * W, 8)
      pltpu.sync_copy(rows2.at[0], acc_sh.at[pl.ds(off, W)])
      pltpu.sync_copy(ones_v, cnt_sh.at[pl.ds(off, W)])
      return 0
    lax.fori_loop(0, NODE_CHUNK // W, zero_chunk, 0)

    def fill_one(i, _):
      ones_v[i] = jnp.ones((16,), jnp.float32)
      return 0
    lax.fori_loop(0, W, fill_one, 0)

    nchunks = CPW + jnp.where(w < CPW_EXTRA, 1, 0)
    base = CPW * w + jnp.minimum(w, CPW_EXTRA)
    pltpu.sync_copy(e3_hbm.at[0, pl.ds(base, CPW)], sidx.at[pl.ds(0, CPW)])
    pltpu.sync_copy(e3_hbm.at[1, pl.ds(base, CPW)], didx.at[pl.ds(0, CPW)])

    @pl.when(w < CPW_EXTRA)
    def _():
      pltpu.sync_copy(e3_hbm.at[0, base + CPW], sidx.at[CPW])
      pltpu.sync_copy(e3_hbm.at[1, base + CPW], didx.at[CPW])

    plsc.subcore_barrier()

    def step(i, carry):
      b = jnp.bitwise_and(i, 1)

      @pl.when(i < nchunks)
      def _():
        pltpu.async_copy(x_hbm.at[sidx.at[i]], rows2.at[b], sem2.at[b])

      @pl.when(i > 0)
      def _():
        pb = jnp.bitwise_and(i - 1, 1)
        pltpu.make_async_copy(
            x_hbm.at[pl.ds(0, W)], rows2.at[pb], sem2.at[pb]).wait()
        pltpu.sync_copy(rows2.at[pb], acc_sh.at[didx.at[i - 1]], add=True)
        pltpu.async_copy(ones_v, cnt_sh.at[didx.at[i - 1]], csem, add=True)
      return carry

    lax.fori_loop(0, nchunks + 1, step, 0)

    def drain(i, carry):
      pltpu.make_async_copy(
          x_hbm.at[pl.ds(0, W), pl.ds(0, 16)], ones_v, csem).wait()
      return carry

    lax.fori_loop(0, nchunks, drain, 0)

    plsc.subcore_barrier()

    pltpu.sync_copy(acc_sh.at[pl.ds(nb, NODE_CHUNK)],
                    acc_out.at[c, pl.ds(nb, NODE_CHUNK)])
    pltpu.sync_copy(cnt_sh.at[pl.ds(nb, NODE_CHUNK)],
                    cnt_out.at[c, pl.ds(nb, NODE_CHUNK)])

  return k(x, e3)


def _tc_body(pacc, pcnt, x, wl, wr, b, out):
  acc = pacc[0] + pacc[1]
  cnt = pcnt[0] + pcnt[1]
  mean = acc / jnp.maximum(cnt[:, 0:1], 1.0)
  y = (jnp.dot(mean, wl[...], preferred_element_type=jnp.float32)
       + b[...]
       + jnp.dot(x[...], wr[...], preferred_element_type=jnp.float32))
  out[...] = jnp.maximum(y, 0.0)


def _tc_finish(pacc, pcnt, x, wlT, wrT, b):
  R = 2000
  grid = (N_NODES // R,)
  return pl.pallas_call(
      _tc_body,
      grid=grid,
      in_specs=[
          pl.BlockSpec((NC, R, C), lambda i: (0, i, 0)),
          pl.BlockSpec((NC, R, 16), lambda i: (0, i, 0)),
          pl.BlockSpec((R, C), lambda i: (i, 0)),
          pl.BlockSpec((C, C), lambda i: (0, 0)),
          pl.BlockSpec((C, C), lambda i: (0, 0)),
          pl.BlockSpec((1, C), lambda i: (0, 0)),
      ],
      out_specs=pl.BlockSpec((R, C), lambda i: (i, 0)),
      out_shape=jax.ShapeDtypeStruct((N_NODES, C), jnp.float32),
  )(pacc, pcnt, x, wlT, wrT, b)


def kernel(x, edge_index, W_l, b_l, W_r):
  e3 = edge_index.reshape(2, NCHUNK, W)
  pacc, pcnt = _sc_aggregate(x, e3)
  return _tc_finish(pacc, pcnt, x, W_l.T, W_r.T, b_l.reshape(1, C))
